# baseline (speedup 1.0000x reference)
import jax
import jax.numpy as jnp
from jax.experimental import pallas as pl

_NODE = 32
_K = 4
_DIMS = _NODE * _K

_IDX = [(37 * t) % _DIMS for t in range(_DIMS)]

_cur = list(range(_DIMS))
for _t in range(_DIMS):
    _cur[_IDX[_t]] = _cur[_t]
_MAP = tuple(_cur)
del _cur, _t


def _affine_flat_kernel(x_ref, a_ref, b_ref, o_ref):
    bb, ff, frame, node = x_ref.shape
    x2 = x_ref[...].reshape(bb * ff, frame * node)
    o_ref[...] = (
        jnp.dot(x2, a_ref[...], preferred_element_type=jnp.float32)
        + b_ref[...]
    )


def kernel(x, W, b):
    batch, features, frame, node = x.shape
    bf = batch * features
    fin = frame * node
    fout = frame * _DIMS

    m = jnp.array(_MAP, dtype=jnp.int32)
    nodei = m // _K
    rem = m % _K
    w_sel = W[nodei]
    b_sel = b[nodei]
    oidx = _K * jnp.arange(frame)[None, :] + rem[:, None]
    w_p = jnp.take_along_axis(w_sel, oidx[:, :, None], axis=1)
    b_p = jnp.take_along_axis(b_sel, oidx, axis=1)
    onehot = (nodei[:, None] == jnp.arange(node)[None, :]).astype(x.dtype)
    a_mat = jnp.einsum("kcd,ki->dick", w_p, onehot).reshape(fin, fout)
    bias = b_p.T.reshape(1, fout)

    bb = 8
    hb = batch // 2

    def run_flat(xh):
        return pl.pallas_call(
            _affine_flat_kernel,
            grid=(hb // bb,),
            in_specs=[
                pl.BlockSpec(
                    (bb, features, frame, node), lambda i: (i, 0, 0, 0)
                ),
                pl.BlockSpec((fin, fout), lambda i: (0, 0)),
                pl.BlockSpec((1, fout), lambda i: (0, 0)),
            ],
            out_specs=pl.BlockSpec((bb * features, fout), lambda i: (i, 0)),
            out_shape=jax.ShapeDtypeStruct((hb * features, fout), x.dtype),
        )(xh, a_mat, bias)

    y1 = run_flat(x[:hb])
    o1 = y1.reshape(hb, features, frame, _DIMS)
    y2 = run_flat(x[hb:])
    o2 = y2.reshape(hb, features, frame, _DIMS)
    return jnp.concatenate([o1, o2], axis=0)


# speedup vs baseline: 2.2074x; 2.2074x over previous
import jax
import jax.numpy as jnp
from jax.experimental import pallas as pl

_NODE = 32
_K = 4
_DIMS = _NODE * _K

_IDX = [(37 * t) % _DIMS for t in range(_DIMS)]

_cur = list(range(_DIMS))
for _t in range(_DIMS):
    _cur[_IDX[_t]] = _cur[_t]
_MAP = tuple(_cur)
del _cur, _t


def _affine_kernel(x_ref, a_ref, b_ref, o_ref):
    bb, ff, frame, _ = o_ref.shape
    y2 = (
        jnp.dot(x_ref[...], a_ref[...], preferred_element_type=jnp.float32)
        + b_ref[...]
    )
    o_ref[...] = y2.reshape(bb, ff, frame, _DIMS)


def kernel(x, W, b):
    batch, features, frame, node = x.shape
    bf = batch * features
    fin = frame * node
    fout = frame * _DIMS

    m = jnp.array(_MAP, dtype=jnp.int32)
    nodei = m // _K
    rem = m % _K
    w_sel = W[nodei]
    b_sel = b[nodei]
    oidx = _K * jnp.arange(frame)[None, :] + rem[:, None]
    w_p = jnp.take_along_axis(w_sel, oidx[:, :, None], axis=1)
    b_p = jnp.take_along_axis(b_sel, oidx, axis=1)
    onehot = (nodei[:, None] == jnp.arange(node)[None, :]).astype(x.dtype)
    a_mat = jnp.einsum("kcd,ki->dick", w_p, onehot).reshape(fin, fout)
    bias = b_p.T.reshape(1, fout)

    bb = 8
    out = pl.pallas_call(
        _affine_kernel,
        grid=(batch // bb,),
        in_specs=[
            pl.BlockSpec((bb * features, fin), lambda i: (i, 0)),
            pl.BlockSpec((fin, fout), lambda i: (0, 0)),
            pl.BlockSpec((1, fout), lambda i: (0, 0)),
        ],
        out_specs=pl.BlockSpec(
            (bb, features, frame, _DIMS), lambda i: (i, 0, 0, 0)
        ),
        out_shape=jax.ShapeDtypeStruct(
            (batch, features, frame, _DIMS), x.dtype
        ),
    )(x.reshape(bf, fin), a_mat, bias)

    return out


# speedup vs baseline: 2.2619x; 1.0247x over previous
import jax
import jax.numpy as jnp
from jax.experimental import pallas as pl

_NODE = 32
_K = 4
_DIMS = _NODE * _K

_IDX = [(37 * t) % _DIMS for t in range(_DIMS)]

_cur = list(range(_DIMS))
for _t in range(_DIMS):
    _cur[_IDX[_t]] = _cur[_t]
_MAP = tuple(_cur)
del _cur, _t


def _affine_kernel(x_ref, a_ref, b_ref, o_ref):
    bb, ff, frame, _ = o_ref.shape
    y2 = (
        jnp.dot(x_ref[...], a_ref[...], preferred_element_type=jnp.float32)
        + b_ref[...]
    )
    o_ref[...] = y2.reshape(bb, ff, frame, _DIMS)


def kernel(x, W, b):
    batch, features, frame, node = x.shape
    bf = batch * features
    fin = frame * node
    fout = frame * _DIMS

    m = jnp.array(_MAP, dtype=jnp.int32)
    nodei = m // _K
    rem = m % _K
    w_sel = W[nodei]
    b_sel = b[nodei]
    oidx = _K * jnp.arange(frame)[None, :] + rem[:, None]
    w_p = jnp.take_along_axis(w_sel, oidx[:, :, None], axis=1)
    b_p = jnp.take_along_axis(b_sel, oidx, axis=1)
    onehot = (nodei[:, None] == jnp.arange(node)[None, :]).astype(x.dtype)
    a_mat = jnp.einsum("kcd,ki->dick", w_p, onehot).reshape(fin, fout)
    bias = b_p.T.reshape(1, fout)

    bb = 16
    out = pl.pallas_call(
        _affine_kernel,
        grid=(batch // bb,),
        in_specs=[
            pl.BlockSpec((bb * features, fin), lambda i: (i, 0)),
            pl.BlockSpec((fin, fout), lambda i: (0, 0)),
            pl.BlockSpec((1, fout), lambda i: (0, 0)),
        ],
        out_specs=pl.BlockSpec(
            (bb, features, frame, _DIMS), lambda i: (i, 0, 0, 0)
        ),
        out_shape=jax.ShapeDtypeStruct(
            (batch, features, frame, _DIMS), x.dtype
        ),
    )(x.reshape(bf, fin), a_mat, bias)

    return out


# speedup vs baseline: 2.2700x; 1.0036x over previous
import jax
import jax.numpy as jnp
from jax.experimental import pallas as pl

_NODE = 32
_K = 4
_DIMS = _NODE * _K

_IDX = [(37 * t) % _DIMS for t in range(_DIMS)]

_cur = list(range(_DIMS))
for _t in range(_DIMS):
    _cur[_IDX[_t]] = _cur[_t]
_MAP = tuple(_cur)
del _cur, _t


def _affine_kernel(x_ref, a_ref, b_ref, o_ref):
    bb, ff, frame, _ = o_ref.shape
    y2 = (
        jnp.dot(x_ref[...], a_ref[...], preferred_element_type=jnp.float32)
        + b_ref[...]
    )
    o_ref[...] = y2.reshape(bb, ff, frame, _DIMS)


def kernel(x, W, b):
    batch, features, frame, node = x.shape
    bf = batch * features
    fin = frame * node
    fout = frame * _DIMS

    m = jnp.array(_MAP, dtype=jnp.int32)
    nodei = m // _K
    rem = m % _K
    w_sel = W[nodei]
    b_sel = b[nodei]
    oidx = _K * jnp.arange(frame)[None, :] + rem[:, None]
    w_p = jnp.take_along_axis(w_sel, oidx[:, :, None], axis=1)
    b_p = jnp.take_along_axis(b_sel, oidx, axis=1)
    onehot = (nodei[:, None] == jnp.arange(node)[None, :]).astype(x.dtype)
    a_mat = jnp.einsum("kcd,ki->dick", w_p, onehot).reshape(fin, fout)
    bias = b_p.T.reshape(1, fout)

    bb = 32
    out = pl.pallas_call(
        _affine_kernel,
        grid=(batch // bb,),
        in_specs=[
            pl.BlockSpec((bb * features, fin), lambda i: (i, 0)),
            pl.BlockSpec((fin, fout), lambda i: (0, 0)),
            pl.BlockSpec((1, fout), lambda i: (0, 0)),
        ],
        out_specs=pl.BlockSpec(
            (bb, features, frame, _DIMS), lambda i: (i, 0, 0, 0)
        ),
        out_shape=jax.ShapeDtypeStruct(
            (batch, features, frame, _DIMS), x.dtype
        ),
    )(x.reshape(bf, fin), a_mat, bias)

    return out
